# baseline (speedup 1.0000x reference)
import functools

import jax
import jax.numpy as jnp
from jax import lax
from jax.experimental import pallas as pl
from jax.experimental.pallas import tpu as pltpu
from jax.experimental.pallas import tpu_sc as plsc

N = 10000
E = 320000
D = 128

NC = 2
NS = 16
NW = NC * NS
CH = 112
NCHUNK = 90
NB = 3
E_PAD = NW * NCHUNK * CH
N_PAD = 10240
ROWS_PT = N_PAD // NS

def _sc_body(y_hbm, src_hbm, dst_hbm, znd_hbm, zdg_hbm, ones_hbm,
             part_hbm, degp_hbm,
             acc_sh, deg_sh, srcs, dsts, rows, ones_v,
             semI, semG, semS, with_deg):
    c = lax.axis_index("c")
    s = lax.axis_index("s")
    wid = s * NC + c
    r0 = s * ROWS_PT
    e_base = wid * (NCHUNK * CH)

    def idx_slice(i):
        return pl.ds(pl.multiple_of(e_base + i * CH, 8), CH)

    pltpu.sync_copy(src_hbm.at[idx_slice(0)], srcs[0])
    pltpu.sync_copy(dst_hbm.at[idx_slice(0)], dsts[0])
    pltpu.async_copy(src_hbm.at[idx_slice(1)], srcs[1], semI[1])
    pltpu.async_copy(dst_hbm.at[idx_slice(1)], dsts[1], semI[1])
    pltpu.async_copy(y_hbm.at[srcs[0]], rows[0], semG[0])

    pltpu.sync_copy(znd_hbm.at[pl.ds(r0, ROWS_PT)], acc_sh.at[pl.ds(r0, ROWS_PT)])
    if with_deg:
        pltpu.sync_copy(zdg_hbm.at[pl.ds(r0, ROWS_PT)], deg_sh.at[pl.ds(r0, ROWS_PT)])
        pltpu.sync_copy(ones_hbm, ones_v)
    plsc.subcore_barrier()

    def _ring_body(j, _):
        for b in range(NB):
            i = j * NB + b
            bm1 = (b - 1) % NB
            bp1 = (b + 1) % NB
            pltpu.make_async_copy(y_hbm.at[srcs[b]], rows[b], semG[b]).wait()
            pltpu.async_copy(rows[b], acc_sh.at[dsts[b]], semS[b], add=True)
            if with_deg:
                pltpu.sync_copy(ones_v, deg_sh.at[dsts[b]], add=True)

            @pl.when(i >= 1)
            def _():
                pltpu.make_async_copy(
                    rows[bm1], acc_sh.at[dsts[bm1]], semS[bm1]).wait()

            @pl.when(i + 2 < NCHUNK)
            def _():
                pltpu.async_copy(src_hbm.at[idx_slice(i + 2)], srcs[bm1], semI[bm1])
                pltpu.async_copy(dst_hbm.at[idx_slice(i + 2)], dsts[bm1], semI[bm1])

            @pl.when(i + 1 < NCHUNK)
            def _():
                pltpu.make_async_copy(
                    src_hbm.at[idx_slice(i + 1)], srcs[bp1], semI[bp1]).wait()
                pltpu.make_async_copy(
                    dst_hbm.at[idx_slice(i + 1)], dsts[bp1], semI[bp1]).wait()
                pltpu.async_copy(y_hbm.at[srcs[bp1]], rows[bp1], semG[bp1])
        return 0

    lax.fori_loop(0, NCHUNK // NB, _ring_body, 0)
    pltpu.make_async_copy(
        rows[(NCHUNK - 1) % NB], acc_sh.at[dsts[(NCHUNK - 1) % NB]],
        semS[(NCHUNK - 1) % NB]).wait()
    plsc.subcore_barrier()

    o0 = c * N_PAD + r0
    pltpu.sync_copy(acc_sh.at[pl.ds(r0, ROWS_PT)], part_hbm.at[pl.ds(o0, ROWS_PT)])
    if with_deg:
        pltpu.sync_copy(deg_sh.at[pl.ds(r0, ROWS_PT)], degp_hbm.at[pl.ds(o0, ROWS_PT)])


@functools.cache
def _sc_kernels():
    mesh = plsc.VectorSubcoreMesh(core_axis_name="c", subcore_axis_name="s")
    idx_scratch = [pltpu.VMEM((CH,), jnp.int32) for _ in range(2 * NB)]
    row_scratch = [pltpu.VMEM((CH, D), jnp.float32) for _ in range(NB)]
    sem_scratch = [pltpu.SemaphoreType.DMA for _ in range(3 * NB)]

    @functools.partial(
        pl.kernel,
        out_type=[jax.ShapeDtypeStruct((2 * N_PAD, D), jnp.float32),
                  jax.ShapeDtypeStruct((2 * N_PAD,), jnp.float32)],
        mesh=mesh,
        scratch_types=[
            pltpu.VMEM_SHARED((N_PAD, D), jnp.float32),
            pltpu.VMEM_SHARED((N_PAD,), jnp.float32),
            *idx_scratch,
            *row_scratch,
            pltpu.VMEM((CH,), jnp.float32),
            *sem_scratch,
        ],
    )
    def sc_agg_deg(y_hbm, src_hbm, dst_hbm, znd_hbm, zdg_hbm, ones_hbm,
                   part_hbm, degp_hbm,
                   acc_sh, deg_sh, s0, s1, s2, d0, d1, d2, r0_, r1_, r2_,
                   ones_v, i0, i1, i2, g0, g1, g2, ss0, ss1, ss2):
        _sc_body(y_hbm, src_hbm, dst_hbm, znd_hbm, zdg_hbm, ones_hbm,
                 part_hbm, degp_hbm,
                 acc_sh, deg_sh, [s0, s1, s2], [d0, d1, d2], [r0_, r1_, r2_],
                 ones_v, [i0, i1, i2], [g0, g1, g2], [ss0, ss1, ss2], True)

    @functools.partial(
        pl.kernel,
        out_type=[jax.ShapeDtypeStruct((2 * N_PAD, D), jnp.float32)],
        mesh=mesh,
        scratch_types=[
            pltpu.VMEM_SHARED((N_PAD, D), jnp.float32),
            *idx_scratch,
            *row_scratch,
            *sem_scratch,
        ],
    )
    def sc_agg(y_hbm, src_hbm, dst_hbm, znd_hbm,
               part_hbm,
               acc_sh, s0, s1, s2, d0, d1, d2, r0_, r1_, r2_,
               i0, i1, i2, g0, g1, g2, ss0, ss1, ss2):
        _sc_body(y_hbm, src_hbm, dst_hbm, znd_hbm, None, None, part_hbm, None,
                 acc_sh, None, [s0, s1, s2], [d0, d1, d2], [r0_, r1_, r2_],
                 None, [i0, i1, i2], [g0, g1, g2], [ss0, ss1, ss2], False)

    return sc_agg_deg, sc_agg


BN = 1000


def _mm2_body(x_ref, wm_ref, wr_ref, y_ref, r_ref):
    xb = x_ref[...]
    y_ref[...] = jnp.dot(xb, wm_ref[...], preferred_element_type=jnp.float32)
    r_ref[...] = jnp.dot(xb, wr_ref[...], preferred_element_type=jnp.float32)


_mm2 = pl.pallas_call(
    _mm2_body,
    grid=(N // BN,),
    in_specs=[pl.BlockSpec((BN, D), lambda i: (i, 0)),
              pl.BlockSpec((D, D), lambda i: (0, 0)),
              pl.BlockSpec((D, D), lambda i: (0, 0))],
    out_specs=[pl.BlockSpec((BN, D), lambda i: (i, 0)),
               pl.BlockSpec((BN, D), lambda i: (i, 0))],
    out_shape=[jax.ShapeDtypeStruct((N, D), jnp.float32),
               jax.ShapeDtypeStruct((N, D), jnp.float32)],
)


def _agg_from_partials(p_ref, dg_ref):
    deg = dg_ref[0] + dg_ref[1]
    invd = 1.0 / jnp.maximum(deg, 1.0)
    return (p_ref[0] + p_ref[1]) * invd


def _comb_body(p_ref, dg_ref, r_ref, b_ref, wm_ref, wr_ref, y2_ref, r2_ref):
    h = jnp.maximum(_agg_from_partials(p_ref, dg_ref) + r_ref[...] + b_ref[...], 0.0)
    y2_ref[...] = jnp.dot(h, wm_ref[...], preferred_element_type=jnp.float32)
    r2_ref[...] = jnp.dot(h, wr_ref[...], preferred_element_type=jnp.float32)


_comb = pl.pallas_call(
    _comb_body,
    grid=(N // BN,),
    in_specs=[pl.BlockSpec((2, BN, D), lambda i: (0, i, 0)),
              pl.BlockSpec((2, BN, 1), lambda i: (0, i, 0)),
              pl.BlockSpec((BN, D), lambda i: (i, 0)),
              pl.BlockSpec((1, D), lambda i: (0, 0)),
              pl.BlockSpec((D, D), lambda i: (0, 0)),
              pl.BlockSpec((D, D), lambda i: (0, 0))],
    out_specs=[pl.BlockSpec((BN, D), lambda i: (i, 0)),
               pl.BlockSpec((BN, D), lambda i: (i, 0))],
    out_shape=[jax.ShapeDtypeStruct((N, D), jnp.float32),
               jax.ShapeDtypeStruct((N, D), jnp.float32)],
)


def _fin_body(p_ref, dg_ref, r_ref, b_ref, o_ref):
    o_ref[...] = jnp.maximum(
        _agg_from_partials(p_ref, dg_ref) + r_ref[...] + b_ref[...], 0.0)


_fin = pl.pallas_call(
    _fin_body,
    grid=(N // BN,),
    in_specs=[pl.BlockSpec((2, BN, D), lambda i: (0, i, 0)),
              pl.BlockSpec((2, BN, 1), lambda i: (0, i, 0)),
              pl.BlockSpec((BN, D), lambda i: (i, 0)),
              pl.BlockSpec((1, D), lambda i: (0, 0))],
    out_specs=pl.BlockSpec((BN, D), lambda i: (i, 0)),
    out_shape=jax.ShapeDtypeStruct((N, D), jnp.float32),
)


def kernel(x, edge_index, W_msg1, W_root1, b1, W_msg2, W_root2, b2):
    pad = E_PAD - E
    pad_i = jnp.arange(pad, dtype=jnp.int32)
    src = jnp.concatenate([edge_index[0], pad_i % N])
    dst = jnp.concatenate([edge_index[1], N + pad_i % (N_PAD - N)])
    znd = jnp.zeros((N_PAD, D), jnp.float32)
    zdg = jnp.zeros((N_PAD,), jnp.float32)
    ones = jnp.ones((CH,), jnp.float32)

    sc_agg_deg, sc_agg = _sc_kernels()
    y1, r1 = _mm2(x, W_msg1, W_root1)
    part1, degp1 = sc_agg_deg(y1, src, dst, znd, zdg, ones)
    p1 = part1.reshape(2, N_PAD, D)
    dg = degp1.reshape(2, N_PAD, 1)
    y2, r2 = _comb(p1, dg, r1, b1.reshape(1, D), W_msg2, W_root2)
    (part2,) = sc_agg(y2, src, dst, znd)
    out = _fin(part2.reshape(2, N_PAD, D), dg, r2, b2.reshape(1, D))
    return out


# speedup vs baseline: 1.0098x; 1.0098x over previous
import functools

import jax
import jax.numpy as jnp
from jax import lax
from jax.experimental import pallas as pl
from jax.experimental.pallas import tpu as pltpu
from jax.experimental.pallas import tpu_sc as plsc

N = 10000
E = 320000
D = 128

NC = 2
NS = 16
NW = NC * NS
CH = 128
NCHUNK = 80
NPH = 2
PH = NCHUNK // NPH
E_PAD = NW * NCHUNK * CH
N_PAD = 10240
ROWS_PT = N_PAD // NS

def _sc_body(y_hbm, src_hbm, dst_hbm, znd_hbm, zdg_hbm, ones_hbm,
             part_hbm, degp_hbm,
             acc_sh, deg_sh, src_all, dst_all, rows0, rows1, ones_v,
             sem0, sem1, ssc0, ssc1, with_deg):
    c = lax.axis_index("c")
    s = lax.axis_index("s")
    wid = s * NC + c
    r0 = s * ROWS_PT

    rows = [rows0, rows1]
    sems = [sem0, sem1]
    sscat = [ssc0, ssc1]
    c0 = pl.multiple_of(wid * NCHUNK, 8)

    pltpu.sync_copy(znd_hbm.at[pl.ds(r0, ROWS_PT)], acc_sh.at[pl.ds(r0, ROWS_PT)])
    if with_deg:
        pltpu.sync_copy(zdg_hbm.at[pl.ds(r0, ROWS_PT)], deg_sh.at[pl.ds(r0, ROWS_PT)])
        pltpu.sync_copy(ones_hbm, ones_v)
    plsc.subcore_barrier()

    for p in range(NPH):
        pltpu.sync_copy(src_hbm.at[pl.ds(c0 + p * PH, PH)], src_all)
        pltpu.sync_copy(dst_hbm.at[pl.ds(c0 + p * PH, PH)], dst_all)
        pltpu.async_copy(y_hbm.at[src_all.at[0]], rows[0], sems[0])

        def _pair_body(j, _):
            for b in range(2):
                i = j * 2 + b
                bm1 = 1 - b
                pltpu.make_async_copy(
                    y_hbm.at[src_all.at[i]], rows[b], sems[b]).wait()
                pltpu.async_copy(rows[b], acc_sh.at[dst_all.at[i]],
                                 sscat[b], add=True)
                if with_deg:
                    pltpu.sync_copy(ones_v, deg_sh.at[dst_all.at[i]], add=True)

                @pl.when(i >= 1)
                def _():
                    pltpu.make_async_copy(
                        rows[bm1], acc_sh.at[dst_all.at[i - 1]],
                        sscat[bm1]).wait()

                @pl.when(i + 1 < PH)
                def _():
                    pltpu.async_copy(
                        y_hbm.at[src_all.at[i + 1]], rows[bm1], sems[bm1])
            return 0

        lax.fori_loop(0, PH // 2, _pair_body, 0)
        pltpu.make_async_copy(
            rows[(PH - 1) % 2], acc_sh.at[dst_all.at[PH - 1]],
            sscat[(PH - 1) % 2]).wait()
    plsc.subcore_barrier()

    o0 = c * N_PAD + r0
    pltpu.sync_copy(acc_sh.at[pl.ds(r0, ROWS_PT)], part_hbm.at[pl.ds(o0, ROWS_PT)])
    if with_deg:
        pltpu.sync_copy(deg_sh.at[pl.ds(r0, ROWS_PT)], degp_hbm.at[pl.ds(o0, ROWS_PT)])


@functools.cache
def _sc_kernels():
    mesh = plsc.VectorSubcoreMesh(core_axis_name="c", subcore_axis_name="s")

    @functools.partial(
        pl.kernel,
        out_type=[jax.ShapeDtypeStruct((2 * N_PAD, D), jnp.float32),
                  jax.ShapeDtypeStruct((2 * N_PAD,), jnp.float32)],
        mesh=mesh,
        scratch_types=[
            pltpu.VMEM_SHARED((N_PAD, D), jnp.float32),
            pltpu.VMEM_SHARED((N_PAD,), jnp.float32),
            pltpu.VMEM((PH, CH), jnp.int32),
            pltpu.VMEM((PH, CH), jnp.int32),
            pltpu.VMEM((CH, D), jnp.float32),
            pltpu.VMEM((CH, D), jnp.float32),
            pltpu.VMEM((CH,), jnp.float32),
            pltpu.SemaphoreType.DMA,
            pltpu.SemaphoreType.DMA,
            pltpu.SemaphoreType.DMA,
            pltpu.SemaphoreType.DMA,
        ],
    )
    def sc_agg_deg(y_hbm, src_hbm, dst_hbm, znd_hbm, zdg_hbm, ones_hbm,
                   part_hbm, degp_hbm,
                   acc_sh, deg_sh, src_all, dst_all, rows0, rows1, ones_v,
                   sem0, sem1, ssc0, ssc1):
        _sc_body(y_hbm, src_hbm, dst_hbm, znd_hbm, zdg_hbm, ones_hbm,
                 part_hbm, degp_hbm,
                 acc_sh, deg_sh, src_all, dst_all, rows0, rows1, ones_v,
                 sem0, sem1, ssc0, ssc1, True)

    @functools.partial(
        pl.kernel,
        out_type=[jax.ShapeDtypeStruct((2 * N_PAD, D), jnp.float32)],
        mesh=mesh,
        scratch_types=[
            pltpu.VMEM_SHARED((N_PAD, D), jnp.float32),
            pltpu.VMEM((PH, CH), jnp.int32),
            pltpu.VMEM((PH, CH), jnp.int32),
            pltpu.VMEM((CH, D), jnp.float32),
            pltpu.VMEM((CH, D), jnp.float32),
            pltpu.SemaphoreType.DMA,
            pltpu.SemaphoreType.DMA,
            pltpu.SemaphoreType.DMA,
            pltpu.SemaphoreType.DMA,
        ],
    )
    def sc_agg(y_hbm, src_hbm, dst_hbm, znd_hbm,
               part_hbm,
               acc_sh, src_all, dst_all, rows0, rows1, sem0, sem1, ssc0, ssc1):
        _sc_body(y_hbm, src_hbm, dst_hbm, znd_hbm, None, None, part_hbm, None,
                 acc_sh, None, src_all, dst_all, rows0, rows1, None,
                 sem0, sem1, ssc0, ssc1, False)

    return sc_agg_deg, sc_agg


BN = 1000


def _mm2_body(x_ref, wm_ref, wr_ref, y_ref, r_ref):
    xb = x_ref[...]
    y_ref[...] = jnp.dot(xb, wm_ref[...], preferred_element_type=jnp.float32)
    r_ref[...] = jnp.dot(xb, wr_ref[...], preferred_element_type=jnp.float32)


_mm2 = pl.pallas_call(
    _mm2_body,
    grid=(N // BN,),
    in_specs=[pl.BlockSpec((BN, D), lambda i: (i, 0)),
              pl.BlockSpec((D, D), lambda i: (0, 0)),
              pl.BlockSpec((D, D), lambda i: (0, 0))],
    out_specs=[pl.BlockSpec((BN, D), lambda i: (i, 0)),
               pl.BlockSpec((BN, D), lambda i: (i, 0))],
    out_shape=[jax.ShapeDtypeStruct((N, D), jnp.float32),
               jax.ShapeDtypeStruct((N, D), jnp.float32)],
)


def _agg_from_partials(p_ref, dg_ref):
    deg = dg_ref[0] + dg_ref[1]
    invd = 1.0 / jnp.maximum(deg, 1.0)
    return (p_ref[0] + p_ref[1]) * invd


def _comb_body(p_ref, dg_ref, r_ref, b_ref, wm_ref, wr_ref, y2_ref, r2_ref):
    h = jnp.maximum(_agg_from_partials(p_ref, dg_ref) + r_ref[...] + b_ref[...], 0.0)
    y2_ref[...] = jnp.dot(h, wm_ref[...], preferred_element_type=jnp.float32)
    r2_ref[...] = jnp.dot(h, wr_ref[...], preferred_element_type=jnp.float32)


_comb = pl.pallas_call(
    _comb_body,
    grid=(N // BN,),
    in_specs=[pl.BlockSpec((2, BN, D), lambda i: (0, i, 0)),
              pl.BlockSpec((2, BN, 1), lambda i: (0, i, 0)),
              pl.BlockSpec((BN, D), lambda i: (i, 0)),
              pl.BlockSpec((1, D), lambda i: (0, 0)),
              pl.BlockSpec((D, D), lambda i: (0, 0)),
              pl.BlockSpec((D, D), lambda i: (0, 0))],
    out_specs=[pl.BlockSpec((BN, D), lambda i: (i, 0)),
               pl.BlockSpec((BN, D), lambda i: (i, 0))],
    out_shape=[jax.ShapeDtypeStruct((N, D), jnp.float32),
               jax.ShapeDtypeStruct((N, D), jnp.float32)],
)


def _fin_body(p_ref, dg_ref, r_ref, b_ref, o_ref):
    o_ref[...] = jnp.maximum(
        _agg_from_partials(p_ref, dg_ref) + r_ref[...] + b_ref[...], 0.0)


_fin = pl.pallas_call(
    _fin_body,
    grid=(N // BN,),
    in_specs=[pl.BlockSpec((2, BN, D), lambda i: (0, i, 0)),
              pl.BlockSpec((2, BN, 1), lambda i: (0, i, 0)),
              pl.BlockSpec((BN, D), lambda i: (i, 0)),
              pl.BlockSpec((1, D), lambda i: (0, 0))],
    out_specs=pl.BlockSpec((BN, D), lambda i: (i, 0)),
    out_shape=jax.ShapeDtypeStruct((N, D), jnp.float32),
)


def kernel(x, edge_index, W_msg1, W_root1, b1, W_msg2, W_root2, b2):
    pad = E_PAD - E
    pad_i = jnp.arange(pad, dtype=jnp.int32)
    src = jnp.concatenate([edge_index[0], pad_i % N]).reshape(NW * NCHUNK, CH)
    dst = jnp.concatenate(
        [edge_index[1], N + pad_i % (N_PAD - N)]).reshape(NW * NCHUNK, CH)
    znd = jnp.zeros((N_PAD, D), jnp.float32)
    zdg = jnp.zeros((N_PAD,), jnp.float32)
    ones = jnp.ones((CH,), jnp.float32)

    sc_agg_deg, sc_agg = _sc_kernels()
    y1, r1 = _mm2(x, W_msg1, W_root1)
    part1, degp1 = sc_agg_deg(y1, src, dst, znd, zdg, ones)
    p1 = part1.reshape(2, N_PAD, D)
    dg = degp1.reshape(2, N_PAD, 1)
    y2, r2 = _comb(p1, dg, r1, b1.reshape(1, D), W_msg2, W_root2)
    (part2,) = sc_agg(y2, src, dst, znd)
    out = _fin(part2.reshape(2, N_PAD, D), dg, r2, b2.reshape(1, D))
    return out


# speedup vs baseline: 1.1689x; 1.1575x over previous
import functools

import jax
import jax.numpy as jnp
from jax import lax
from jax.experimental import pallas as pl
from jax.experimental.pallas import tpu as pltpu
from jax.experimental.pallas import tpu_sc as plsc

N = 10000
E = 320000
D = 128

NC = 2
NS = 16
NW = NC * NS
CH = 128
NCHUNK = 80
NPH = 2
PH = NCHUNK // NPH
E_PAD = NW * NCHUNK * CH
N_PAD = 10240
ROWS_PT = N_PAD // NS

def _sc_body(y_hbm, src_hbm, dst_hbm, znd_hbm, zdg_hbm, ones_hbm,
             part_hbm, degp_hbm,
             acc_sh, deg_sh, src_all, dst_all, rows0, rows1, ones_v,
             sem0, sem1, with_deg):
    c = lax.axis_index("c")
    s = lax.axis_index("s")
    wid = s * NC + c
    r0 = s * ROWS_PT

    rows = [rows0, rows1]
    sems = [sem0, sem1]
    c0 = pl.multiple_of(wid * NCHUNK, 8)

    pltpu.sync_copy(znd_hbm.at[pl.ds(r0, ROWS_PT)], acc_sh.at[pl.ds(r0, ROWS_PT)])
    if with_deg:
        pltpu.sync_copy(zdg_hbm.at[pl.ds(r0, ROWS_PT)], deg_sh.at[pl.ds(r0, ROWS_PT)])
        pltpu.sync_copy(ones_hbm, ones_v)
    plsc.subcore_barrier()

    for p in range(NPH):
        pltpu.sync_copy(src_hbm.at[pl.ds(c0 + p * PH, PH)], src_all)
        pltpu.sync_copy(dst_hbm.at[pl.ds(c0 + p * PH, PH)], dst_all)
        pltpu.async_copy(y_hbm.at[src_all.at[0]], rows[0], sems[0])
        pltpu.async_copy(y_hbm.at[src_all.at[1]], rows[1], sems[1])

        def _pair_body(j, _):
            for b in range(2):
                i = j * 2 + b
                pltpu.make_async_copy(
                    y_hbm.at[src_all.at[i]], rows[b], sems[b]).wait()
                pltpu.sync_copy(rows[b], acc_sh.at[dst_all.at[i]], add=True)
                if with_deg:
                    pltpu.sync_copy(ones_v, deg_sh.at[dst_all.at[i]], add=True)

                @pl.when(i + 2 < PH)
                def _():
                    pltpu.async_copy(y_hbm.at[src_all.at[i + 2]], rows[b], sems[b])
            return 0

        lax.fori_loop(0, PH // 2, _pair_body, 0)
    plsc.subcore_barrier()

    o0 = c * N_PAD + r0
    pltpu.sync_copy(acc_sh.at[pl.ds(r0, ROWS_PT)], part_hbm.at[pl.ds(o0, ROWS_PT)])
    if with_deg:
        pltpu.sync_copy(deg_sh.at[pl.ds(r0, ROWS_PT)], degp_hbm.at[pl.ds(o0, ROWS_PT)])


@functools.cache
def _sc_kernels():
    mesh = plsc.VectorSubcoreMesh(core_axis_name="c", subcore_axis_name="s")

    @functools.partial(
        pl.kernel,
        out_type=[jax.ShapeDtypeStruct((2 * N_PAD, D), jnp.float32),
                  jax.ShapeDtypeStruct((2 * N_PAD,), jnp.float32)],
        mesh=mesh,
        scratch_types=[
            pltpu.VMEM_SHARED((N_PAD, D), jnp.float32),
            pltpu.VMEM_SHARED((N_PAD,), jnp.float32),
            pltpu.VMEM((PH, CH), jnp.int32),
            pltpu.VMEM((PH, CH), jnp.int32),
            pltpu.VMEM((CH, D), jnp.float32),
            pltpu.VMEM((CH, D), jnp.float32),
            pltpu.VMEM((CH,), jnp.float32),
            pltpu.SemaphoreType.DMA,
            pltpu.SemaphoreType.DMA,
        ],
    )
    def sc_agg_deg(y_hbm, src_hbm, dst_hbm, znd_hbm, zdg_hbm, ones_hbm,
                   part_hbm, degp_hbm,
                   acc_sh, deg_sh, src_all, dst_all, rows0, rows1, ones_v,
                   sem0, sem1):
        _sc_body(y_hbm, src_hbm, dst_hbm, znd_hbm, zdg_hbm, ones_hbm,
                 part_hbm, degp_hbm,
                 acc_sh, deg_sh, src_all, dst_all, rows0, rows1, ones_v,
                 sem0, sem1, True)

    @functools.partial(
        pl.kernel,
        out_type=[jax.ShapeDtypeStruct((2 * N_PAD, D), jnp.float32)],
        mesh=mesh,
        scratch_types=[
            pltpu.VMEM_SHARED((N_PAD, D), jnp.float32),
            pltpu.VMEM((PH, CH), jnp.int32),
            pltpu.VMEM((PH, CH), jnp.int32),
            pltpu.VMEM((CH, D), jnp.float32),
            pltpu.VMEM((CH, D), jnp.float32),
            pltpu.SemaphoreType.DMA,
            pltpu.SemaphoreType.DMA,
        ],
    )
    def sc_agg(y_hbm, src_hbm, dst_hbm, znd_hbm,
               part_hbm,
               acc_sh, src_all, dst_all, rows0, rows1, sem0, sem1):
        _sc_body(y_hbm, src_hbm, dst_hbm, znd_hbm, None, None, part_hbm, None,
                 acc_sh, None, src_all, dst_all, rows0, rows1, None,
                 sem0, sem1, False)

    return sc_agg_deg, sc_agg


BN = 1000


def _mm2_body(x_ref, wm_ref, wr_ref, y_ref, r_ref):
    xb = x_ref[...]
    y_ref[...] = jnp.dot(xb, wm_ref[...], preferred_element_type=jnp.float32)
    r_ref[...] = jnp.dot(xb, wr_ref[...], preferred_element_type=jnp.float32)


_mm2 = pl.pallas_call(
    _mm2_body,
    grid=(N // BN,),
    in_specs=[pl.BlockSpec((BN, D), lambda i: (i, 0)),
              pl.BlockSpec((D, D), lambda i: (0, 0)),
              pl.BlockSpec((D, D), lambda i: (0, 0))],
    out_specs=[pl.BlockSpec((BN, D), lambda i: (i, 0)),
               pl.BlockSpec((BN, D), lambda i: (i, 0))],
    out_shape=[jax.ShapeDtypeStruct((N, D), jnp.float32),
               jax.ShapeDtypeStruct((N, D), jnp.float32)],
)


def _agg_from_partials(p_ref, dg_ref):
    deg = dg_ref[0] + dg_ref[1]
    invd = 1.0 / jnp.maximum(deg, 1.0)
    return (p_ref[0] + p_ref[1]) * invd


def _comb_body(p_ref, dg_ref, r_ref, b_ref, wm_ref, wr_ref, y2_ref, r2_ref):
    h = jnp.maximum(_agg_from_partials(p_ref, dg_ref) + r_ref[...] + b_ref[...], 0.0)
    y2_ref[...] = jnp.dot(h, wm_ref[...], preferred_element_type=jnp.float32)
    r2_ref[...] = jnp.dot(h, wr_ref[...], preferred_element_type=jnp.float32)


_comb = pl.pallas_call(
    _comb_body,
    grid=(N // BN,),
    in_specs=[pl.BlockSpec((2, BN, D), lambda i: (0, i, 0)),
              pl.BlockSpec((2, BN, 1), lambda i: (0, i, 0)),
              pl.BlockSpec((BN, D), lambda i: (i, 0)),
              pl.BlockSpec((1, D), lambda i: (0, 0)),
              pl.BlockSpec((D, D), lambda i: (0, 0)),
              pl.BlockSpec((D, D), lambda i: (0, 0))],
    out_specs=[pl.BlockSpec((BN, D), lambda i: (i, 0)),
               pl.BlockSpec((BN, D), lambda i: (i, 0))],
    out_shape=[jax.ShapeDtypeStruct((N, D), jnp.float32),
               jax.ShapeDtypeStruct((N, D), jnp.float32)],
)


def _fin_body(p_ref, dg_ref, r_ref, b_ref, o_ref):
    o_ref[...] = jnp.maximum(
        _agg_from_partials(p_ref, dg_ref) + r_ref[...] + b_ref[...], 0.0)


_fin = pl.pallas_call(
    _fin_body,
    grid=(N // BN,),
    in_specs=[pl.BlockSpec((2, BN, D), lambda i: (0, i, 0)),
              pl.BlockSpec((2, BN, 1), lambda i: (0, i, 0)),
              pl.BlockSpec((BN, D), lambda i: (i, 0)),
              pl.BlockSpec((1, D), lambda i: (0, 0))],
    out_specs=pl.BlockSpec((BN, D), lambda i: (i, 0)),
    out_shape=jax.ShapeDtypeStruct((N, D), jnp.float32),
)


def kernel(x, edge_index, W_msg1, W_root1, b1, W_msg2, W_root2, b2):
    pad = E_PAD - E
    pad_blk = jnp.tile(jnp.arange(N_PAD - N, dtype=jnp.int32),
                       pad // (N_PAD - N))
    src = jnp.concatenate([edge_index[0], pad_blk]).reshape(NW * NCHUNK, CH)
    dst = jnp.concatenate([edge_index[1], N + pad_blk]).reshape(NW * NCHUNK, CH)
    znd = jnp.zeros((N_PAD, D), jnp.float32)
    zdg = jnp.zeros((N_PAD,), jnp.float32)
    ones = jnp.ones((CH,), jnp.float32)

    sc_agg_deg, sc_agg = _sc_kernels()
    y1, r1 = _mm2(x, W_msg1, W_root1)
    part1, degp1 = sc_agg_deg(y1, src, dst, znd, zdg, ones)
    p1 = part1.reshape(2, N_PAD, D)
    dg = degp1.reshape(2, N_PAD, 1)
    y2, r2 = _comb(p1, dg, r1, b1.reshape(1, D), W_msg2, W_root2)
    (part2,) = sc_agg(y2, src, dst, znd)
    out = _fin(part2.reshape(2, N_PAD, D), dg, r2, b2.reshape(1, D))
    return out


# speedup vs baseline: 1.2069x; 1.0326x over previous
import functools

import jax
import jax.numpy as jnp
from jax import lax
from jax.experimental import pallas as pl
from jax.experimental.pallas import tpu as pltpu
from jax.experimental.pallas import tpu_sc as plsc

N = 10000
E = 320000
D = 128

NC = 2
NS = 16
NW = NC * NS
CH = 128
NCHUNK = 80
NPH = 2
PH = NCHUNK // NPH
E_PAD = NW * NCHUNK * CH
N_PAD = 10240
ROWS_PT = N_PAD // NS

def _sc_body(y_hbm, src_hbm, dst_hbm, ones_hbm,
             part_hbm, degp_hbm,
             acc_sh, deg_sh, src_all, dst_all, rows0, rows1, ones_v,
             sem0, sem1, with_deg):
    c = lax.axis_index("c")
    s = lax.axis_index("s")
    wid = s * NC + c
    r0 = s * ROWS_PT

    rows = [rows0, rows1]
    sems = [sem0, sem1]
    c0 = pl.multiple_of(wid * NCHUNK, 8)

    def _zr(i, _):
        for jj in range(D // 16):
            rows0[i, pl.ds(jj * 16, 16)] = jnp.zeros((16,), jnp.float32)
        return 0

    lax.fori_loop(0, CH, _zr, 0)
    for k in range(ROWS_PT // CH):
        pltpu.sync_copy(rows0, acc_sh.at[pl.ds(r0 + k * CH, CH)])
    if with_deg:
        for k in range(ROWS_PT // CH):
            pltpu.sync_copy(rows0.at[0], deg_sh.at[pl.ds(r0 + k * CH, CH)])
        pltpu.sync_copy(ones_hbm, ones_v)
    plsc.subcore_barrier()

    for p in range(NPH):
        pltpu.sync_copy(src_hbm.at[pl.ds(c0 + p * PH, PH)], src_all)
        pltpu.sync_copy(dst_hbm.at[pl.ds(c0 + p * PH, PH)], dst_all)
        pltpu.async_copy(y_hbm.at[src_all.at[0]], rows[0], sems[0])
        pltpu.async_copy(y_hbm.at[src_all.at[1]], rows[1], sems[1])


        def _pair_body(j, _):
            for b in range(2):
                i = j * 2 + b
                pltpu.make_async_copy(
                    y_hbm.at[src_all.at[i]], rows[b], sems[b]).wait()
                pltpu.sync_copy(rows[b], acc_sh.at[dst_all.at[i]], add=True)
                if with_deg:
                    pltpu.sync_copy(ones_v, deg_sh.at[dst_all.at[i]], add=True)

                @pl.when(i + 2 < PH)
                def _():
                    pltpu.async_copy(y_hbm.at[src_all.at[i + 2]], rows[b], sems[b])
            return 0

        lax.fori_loop(0, PH // 2, _pair_body, 0)
    plsc.subcore_barrier()

    o0 = c * N_PAD + r0
    pltpu.sync_copy(acc_sh.at[pl.ds(r0, ROWS_PT)], part_hbm.at[pl.ds(o0, ROWS_PT)])
    if with_deg:
        pltpu.sync_copy(deg_sh.at[pl.ds(r0, ROWS_PT)], degp_hbm.at[pl.ds(o0, ROWS_PT)])


@functools.cache
def _sc_kernels():
    mesh = plsc.VectorSubcoreMesh(core_axis_name="c", subcore_axis_name="s")

    @functools.partial(
        pl.kernel,
        out_type=[jax.ShapeDtypeStruct((2 * N_PAD, D), jnp.float32),
                  jax.ShapeDtypeStruct((2 * N_PAD,), jnp.float32)],
        mesh=mesh,
        scratch_types=[
            pltpu.VMEM_SHARED((N_PAD, D), jnp.float32),
            pltpu.VMEM_SHARED((N_PAD,), jnp.float32),
            pltpu.VMEM((PH, CH), jnp.int32),
            pltpu.VMEM((PH, CH), jnp.int32),
            pltpu.VMEM((CH, D), jnp.float32),
            pltpu.VMEM((CH, D), jnp.float32),
            pltpu.VMEM((CH,), jnp.float32),
            pltpu.SemaphoreType.DMA,
            pltpu.SemaphoreType.DMA,
        ],
    )
    def sc_agg_deg(y_hbm, src_hbm, dst_hbm, ones_hbm,
                   part_hbm, degp_hbm,
                   acc_sh, deg_sh, src_all, dst_all, rows0, rows1, ones_v,
                   sem0, sem1):
        _sc_body(y_hbm, src_hbm, dst_hbm, ones_hbm,
                 part_hbm, degp_hbm,
                 acc_sh, deg_sh, src_all, dst_all, rows0, rows1, ones_v,
                 sem0, sem1, True)

    @functools.partial(
        pl.kernel,
        out_type=[jax.ShapeDtypeStruct((2 * N_PAD, D), jnp.float32)],
        mesh=mesh,
        scratch_types=[
            pltpu.VMEM_SHARED((N_PAD, D), jnp.float32),
            pltpu.VMEM((PH, CH), jnp.int32),
            pltpu.VMEM((PH, CH), jnp.int32),
            pltpu.VMEM((CH, D), jnp.float32),
            pltpu.VMEM((CH, D), jnp.float32),
            pltpu.SemaphoreType.DMA,
            pltpu.SemaphoreType.DMA,
        ],
    )
    def sc_agg(y_hbm, src_hbm, dst_hbm,
               part_hbm,
               acc_sh, src_all, dst_all, rows0, rows1, sem0, sem1):
        _sc_body(y_hbm, src_hbm, dst_hbm, None, part_hbm, None,
                 acc_sh, None, src_all, dst_all, rows0, rows1, None,
                 sem0, sem1, False)

    return sc_agg_deg, sc_agg


BN = 1000


def _mm2_body(x_ref, wm_ref, wr_ref, y_ref, r_ref):
    xb = x_ref[...]
    y_ref[...] = jnp.dot(xb, wm_ref[...], preferred_element_type=jnp.float32)
    r_ref[...] = jnp.dot(xb, wr_ref[...], preferred_element_type=jnp.float32)


_mm2 = pl.pallas_call(
    _mm2_body,
    grid=(N // BN,),
    in_specs=[pl.BlockSpec((BN, D), lambda i: (i, 0)),
              pl.BlockSpec((D, D), lambda i: (0, 0)),
              pl.BlockSpec((D, D), lambda i: (0, 0))],
    out_specs=[pl.BlockSpec((BN, D), lambda i: (i, 0)),
               pl.BlockSpec((BN, D), lambda i: (i, 0))],
    out_shape=[jax.ShapeDtypeStruct((N, D), jnp.float32),
               jax.ShapeDtypeStruct((N, D), jnp.float32)],
)


def _agg_from_partials(p_ref, dg_ref):
    deg = dg_ref[0] + dg_ref[1]
    invd = 1.0 / jnp.maximum(deg, 1.0)
    return (p_ref[0] + p_ref[1]) * invd


def _comb_body(p_ref, dg_ref, r_ref, b_ref, wm_ref, wr_ref, y2_ref, r2_ref):
    h = jnp.maximum(_agg_from_partials(p_ref, dg_ref) + r_ref[...] + b_ref[...], 0.0)
    y2_ref[...] = jnp.dot(h, wm_ref[...], preferred_element_type=jnp.float32)
    r2_ref[...] = jnp.dot(h, wr_ref[...], preferred_element_type=jnp.float32)


_comb = pl.pallas_call(
    _comb_body,
    grid=(N // BN,),
    in_specs=[pl.BlockSpec((2, BN, D), lambda i: (0, i, 0)),
              pl.BlockSpec((2, BN, 1), lambda i: (0, i, 0)),
              pl.BlockSpec((BN, D), lambda i: (i, 0)),
              pl.BlockSpec((1, D), lambda i: (0, 0)),
              pl.BlockSpec((D, D), lambda i: (0, 0)),
              pl.BlockSpec((D, D), lambda i: (0, 0))],
    out_specs=[pl.BlockSpec((BN, D), lambda i: (i, 0)),
               pl.BlockSpec((BN, D), lambda i: (i, 0))],
    out_shape=[jax.ShapeDtypeStruct((N, D), jnp.float32),
               jax.ShapeDtypeStruct((N, D), jnp.float32)],
)


def _fin_body(p_ref, dg_ref, r_ref, b_ref, o_ref):
    o_ref[...] = jnp.maximum(
        _agg_from_partials(p_ref, dg_ref) + r_ref[...] + b_ref[...], 0.0)


_fin = pl.pallas_call(
    _fin_body,
    grid=(N // BN,),
    in_specs=[pl.BlockSpec((2, BN, D), lambda i: (0, i, 0)),
              pl.BlockSpec((2, BN, 1), lambda i: (0, i, 0)),
              pl.BlockSpec((BN, D), lambda i: (i, 0)),
              pl.BlockSpec((1, D), lambda i: (0, 0))],
    out_specs=pl.BlockSpec((BN, D), lambda i: (i, 0)),
    out_shape=jax.ShapeDtypeStruct((N, D), jnp.float32),
)


def kernel(x, edge_index, W_msg1, W_root1, b1, W_msg2, W_root2, b2):
    pad_rows = (E_PAD - E) // CH
    src2 = jax.lax.slice(edge_index, (0, 0), (1, E)).reshape(E // CH, CH)
    dst2 = jax.lax.slice(edge_index, (1, 0), (2, E)).reshape(E // CH, CH)
    pad_iota = jax.lax.broadcasted_iota(jnp.int32, (pad_rows, CH), 1)
    src = jnp.concatenate([src2, pad_iota], axis=0)
    dst = jnp.concatenate([dst2, N + pad_iota], axis=0)
    ones = jnp.ones((CH,), jnp.float32)

    sc_agg_deg, sc_agg = _sc_kernels()
    y1, r1 = _mm2(x, W_msg1, W_root1)
    part1, degp1 = sc_agg_deg(y1, src, dst, ones)
    p1 = part1.reshape(2, N_PAD, D)
    dg = degp1.reshape(2, N_PAD, 1)
    y2, r2 = _comb(p1, dg, r1, b1.reshape(1, D), W_msg2, W_root2)
    (part2,) = sc_agg(y2, src, dst)
    out = _fin(part2.reshape(2, N_PAD, D), dg, r2, b2.reshape(1, D))
    return out


# speedup vs baseline: 1.2542x; 1.0392x over previous
import functools

import jax
import jax.numpy as jnp
from jax import lax
from jax.experimental import pallas as pl
from jax.experimental.pallas import tpu as pltpu
from jax.experimental.pallas import tpu_sc as plsc

N = 10000
E = 320000
D = 128

NC = 2
NS = 16
NW = NC * NS
CH = 128
NCHUNK = 80
NPH = 2
PH = NCHUNK // NPH
E_PAD = NW * NCHUNK * CH
N_PAD = 10240
ROWS_PT = N_PAD // NS

def _sc_body(y_hbm, src_hbm, dst_hbm, ones_hbm,
             part_hbm, degp_hbm,
             acc_sh, deg_sh, src_all, dst_all, rows0, rows1, ones_v,
             sem0, sem1, with_deg):
    c = lax.axis_index("c")
    s = lax.axis_index("s")
    wid = s * NC + c
    r0 = s * ROWS_PT

    rows = [rows0, rows1]
    sems = [sem0, sem1]
    c0 = pl.multiple_of(wid * NCHUNK, 8)

    def _zr(i, _):
        for jj in range(D // 16):
            rows0[i, pl.ds(jj * 16, 16)] = jnp.zeros((16,), jnp.float32)
        return 0

    lax.fori_loop(0, CH, _zr, 0)
    for k in range(ROWS_PT // CH):
        pltpu.sync_copy(rows0, acc_sh.at[pl.ds(r0 + k * CH, CH)])
    if with_deg:
        for k in range(ROWS_PT // CH):
            pltpu.sync_copy(rows0.at[0], deg_sh.at[pl.ds(r0 + k * CH, CH)])
        pltpu.sync_copy(ones_hbm, ones_v)
    plsc.subcore_barrier()

    for p in range(NPH):
        pltpu.sync_copy(src_hbm.at[pl.ds(c0 + p * PH, PH)], src_all)
        pltpu.sync_copy(dst_hbm.at[pl.ds(c0 + p * PH, PH)], dst_all)
        pltpu.async_copy(y_hbm.at[src_all.at[0]], rows[0], sems[0])
        pltpu.async_copy(y_hbm.at[src_all.at[1]], rows[1], sems[1])


        def _pair_body(j, _):
            for b in range(2):
                i = j * 2 + b
                pltpu.make_async_copy(
                    y_hbm.at[src_all.at[i]], rows[b], sems[b]).wait()
                pltpu.sync_copy(rows[b], acc_sh.at[dst_all.at[i]], add=True)
                if with_deg:
                    pltpu.sync_copy(ones_v, deg_sh.at[dst_all.at[i]], add=True)

                @pl.when(i + 2 < PH)
                def _():
                    pltpu.async_copy(y_hbm.at[src_all.at[i + 2]], rows[b], sems[b])
            return 0

        lax.fori_loop(0, PH // 2, _pair_body, 0)
    plsc.subcore_barrier()

    o0 = c * N_PAD + r0
    pltpu.sync_copy(acc_sh.at[pl.ds(r0, ROWS_PT)], part_hbm.at[pl.ds(o0, ROWS_PT)])
    if with_deg:
        pltpu.sync_copy(deg_sh.at[pl.ds(r0, ROWS_PT)], degp_hbm.at[pl.ds(o0, ROWS_PT)])


@functools.cache
def _sc_kernels():
    mesh = plsc.VectorSubcoreMesh(core_axis_name="c", subcore_axis_name="s")

    @functools.partial(
        pl.kernel,
        out_type=[jax.ShapeDtypeStruct((2 * N_PAD, D), jnp.float32),
                  jax.ShapeDtypeStruct((2 * N_PAD,), jnp.float32)],
        mesh=mesh,
        scratch_types=[
            pltpu.VMEM_SHARED((N_PAD, D), jnp.float32),
            pltpu.VMEM_SHARED((N_PAD,), jnp.float32),
            pltpu.VMEM((PH, CH), jnp.int32),
            pltpu.VMEM((PH, CH), jnp.int32),
            pltpu.VMEM((CH, D), jnp.float32),
            pltpu.VMEM((CH, D), jnp.float32),
            pltpu.VMEM((CH,), jnp.float32),
            pltpu.SemaphoreType.DMA,
            pltpu.SemaphoreType.DMA,
        ],
    )
    def sc_agg_deg(y_hbm, src_hbm, dst_hbm, ones_hbm,
                   part_hbm, degp_hbm,
                   acc_sh, deg_sh, src_all, dst_all, rows0, rows1, ones_v,
                   sem0, sem1):
        _sc_body(y_hbm, src_hbm, dst_hbm, ones_hbm,
                 part_hbm, degp_hbm,
                 acc_sh, deg_sh, src_all, dst_all, rows0, rows1, ones_v,
                 sem0, sem1, True)

    @functools.partial(
        pl.kernel,
        out_type=[jax.ShapeDtypeStruct((2 * N_PAD, D), jnp.float32)],
        mesh=mesh,
        scratch_types=[
            pltpu.VMEM_SHARED((N_PAD, D), jnp.float32),
            pltpu.VMEM((PH, CH), jnp.int32),
            pltpu.VMEM((PH, CH), jnp.int32),
            pltpu.VMEM((CH, D), jnp.float32),
            pltpu.VMEM((CH, D), jnp.float32),
            pltpu.SemaphoreType.DMA,
            pltpu.SemaphoreType.DMA,
        ],
    )
    def sc_agg(y_hbm, src_hbm, dst_hbm,
               part_hbm,
               acc_sh, src_all, dst_all, rows0, rows1, sem0, sem1):
        _sc_body(y_hbm, src_hbm, dst_hbm, None, part_hbm, None,
                 acc_sh, None, src_all, dst_all, rows0, rows1, None,
                 sem0, sem1, False)

    return sc_agg_deg, sc_agg


BN = 1000
BNP = 1024


def _mm2_body(x_ref, wm_ref, wr_ref, y_ref, r_ref):
    xb = x_ref[...]
    y_ref[...] = jnp.dot(xb, wm_ref[...], preferred_element_type=jnp.float32)
    r_ref[...] = jnp.dot(xb, wr_ref[...], preferred_element_type=jnp.float32)


_mm2 = pl.pallas_call(
    _mm2_body,
    grid=(N // BN,),
    in_specs=[pl.BlockSpec((BN, D), lambda i: (i, 0)),
              pl.BlockSpec((D, D), lambda i: (0, 0)),
              pl.BlockSpec((D, D), lambda i: (0, 0))],
    out_specs=[pl.BlockSpec((BN, D), lambda i: (i, 0)),
               pl.BlockSpec((BN, D), lambda i: (i, 0))],
    out_shape=[jax.ShapeDtypeStruct((N_PAD, D), jnp.float32),
               jax.ShapeDtypeStruct((N_PAD, D), jnp.float32)],
)


def _comb_body(p_ref, dg0_ref, dg1_ref, r_ref, b_ref, wm_ref, wr_ref,
               y2_ref, r2_ref, invb_ref):
    invd_row = 1.0 / jnp.maximum(dg0_ref[...] + dg1_ref[...], 1.0)
    invd = jnp.swapaxes(invd_row, 0, 1)
    h = jnp.maximum((p_ref[0] + p_ref[1]) * invd + r_ref[...] + b_ref[...],
                    0.0)
    y2_ref[...] = jnp.dot(h, wm_ref[...], preferred_element_type=jnp.float32)
    r2_ref[...] = jnp.dot(h, wr_ref[...], preferred_element_type=jnp.float32)
    invb_ref[...] = jnp.broadcast_to(invd, (BNP, D))


_comb = pl.pallas_call(
    _comb_body,
    grid=(N_PAD // BNP,),
    in_specs=[pl.BlockSpec((2, BNP, D), lambda i: (0, i, 0)),
              pl.BlockSpec((1, BNP), lambda i: (0, i)),
              pl.BlockSpec((1, BNP), lambda i: (0, i)),
              pl.BlockSpec((BNP, D), lambda i: (i, 0)),
              pl.BlockSpec((1, D), lambda i: (0, 0)),
              pl.BlockSpec((D, D), lambda i: (0, 0)),
              pl.BlockSpec((D, D), lambda i: (0, 0))],
    out_specs=[pl.BlockSpec((BNP, D), lambda i: (i, 0)),
               pl.BlockSpec((BNP, D), lambda i: (i, 0)),
               pl.BlockSpec((BNP, D), lambda i: (i, 0))],
    out_shape=[jax.ShapeDtypeStruct((N_PAD, D), jnp.float32),
               jax.ShapeDtypeStruct((N_PAD, D), jnp.float32),
               jax.ShapeDtypeStruct((N_PAD, D), jnp.float32)],
)


def _fin_body(p_ref, invb_ref, r_ref, b_ref, o_ref):
    o_ref[...] = jnp.maximum(
        (p_ref[0] + p_ref[1]) * invb_ref[...] + r_ref[...] + b_ref[...], 0.0)


_fin = pl.pallas_call(
    _fin_body,
    grid=(N // BN,),
    in_specs=[pl.BlockSpec((2, BN, D), lambda i: (0, i, 0)),
              pl.BlockSpec((BN, D), lambda i: (i, 0)),
              pl.BlockSpec((BN, D), lambda i: (i, 0)),
              pl.BlockSpec((1, D), lambda i: (0, 0))],
    out_specs=pl.BlockSpec((BN, D), lambda i: (i, 0)),
    out_shape=jax.ShapeDtypeStruct((N, D), jnp.float32),
)


def kernel(x, edge_index, W_msg1, W_root1, b1, W_msg2, W_root2, b2):
    pad_rows = (E_PAD - E) // CH
    src2 = jax.lax.slice(edge_index, (0, 0), (1, E)).reshape(E // CH, CH)
    dst2 = jax.lax.slice(edge_index, (1, 0), (2, E)).reshape(E // CH, CH)
    pad_iota = jax.lax.broadcasted_iota(jnp.int32, (pad_rows, CH), 1)
    src = jnp.concatenate([src2, pad_iota], axis=0)
    dst = jnp.concatenate([dst2, N + pad_iota], axis=0)
    ones = jnp.ones((CH,), jnp.float32)

    sc_agg_deg, sc_agg = _sc_kernels()
    y1, r1 = _mm2(x, W_msg1, W_root1)
    part1, degp1 = sc_agg_deg(y1, src, dst, ones)
    p1 = part1.reshape(2, N_PAD, D)
    dg0 = degp1[:N_PAD].reshape(1, N_PAD)
    dg1 = degp1[N_PAD:].reshape(1, N_PAD)
    y2, r2, invb = _comb(p1, dg0, dg1, r1, b1.reshape(1, D), W_msg2, W_root2)
    (part2,) = sc_agg(y2, src, dst)
    out = _fin(part2.reshape(2, N_PAD, D), invb, r2, b2.reshape(1, D))
    return out


# speedup vs baseline: 1.2945x; 1.0321x over previous
import functools

import jax
import jax.numpy as jnp
from jax import lax
from jax.experimental import pallas as pl
from jax.experimental.pallas import tpu as pltpu
from jax.experimental.pallas import tpu_sc as plsc

N = 10000
E = 320000
D = 128

NC = 2
NS = 16
NW = NC * NS
CH = 128
NCHUNK = 80
NPH = 2
PH = NCHUNK // NPH
E_PAD = NW * NCHUNK * CH
N_PAD = 10240
ROWS_PT = N_PAD // NS

def _sc_body(y_hbm, src_hbm, dst_hbm, ones_hbm,
             part_hbm, degp_hbm,
             acc_sh, deg_sh, src_all, dst_all, rows0, rows1, ones_v,
             sem0, sem1, with_deg):
    c = lax.axis_index("c")
    s = lax.axis_index("s")
    wid = s * NC + c
    r0 = s * ROWS_PT

    rows = [rows0, rows1]
    sems = [sem0, sem1]
    c0 = pl.multiple_of(wid * NCHUNK, 8)

    def _zr(i, _):
        for jj in range(D // 16):
            rows0[i, pl.ds(jj * 16, 16)] = jnp.zeros((16,), jnp.float32)
        return 0

    lax.fori_loop(0, CH, _zr, 0)
    for k in range(ROWS_PT // CH):
        pltpu.sync_copy(rows0, acc_sh.at[pl.ds(r0 + k * CH, CH)])
    if with_deg:
        for k in range(ROWS_PT // CH):
            pltpu.sync_copy(rows0.at[0], deg_sh.at[pl.ds(r0 + k * CH, CH)])
        pltpu.sync_copy(ones_hbm, ones_v)
    plsc.subcore_barrier()

    for p in range(NPH):
        pltpu.sync_copy(src_hbm.at[pl.ds(c0 + p * PH, PH)], src_all)
        pltpu.sync_copy(dst_hbm.at[pl.ds(c0 + p * PH, PH)], dst_all)
        pltpu.async_copy(y_hbm.at[src_all.at[0]], rows[0], sems[0])
        pltpu.async_copy(y_hbm.at[src_all.at[1]], rows[1], sems[1])


        def _pair_body(j, _):
            for b in range(2):
                i = j * 2 + b
                pltpu.make_async_copy(
                    y_hbm.at[src_all.at[i]], rows[b], sems[b]).wait()
                pltpu.sync_copy(rows[b], acc_sh.at[dst_all.at[i]], add=True)
                if with_deg:
                    pltpu.sync_copy(ones_v, deg_sh.at[dst_all.at[i]], add=True)

                @pl.when(i + 2 < PH)
                def _():
                    pltpu.async_copy(y_hbm.at[src_all.at[i + 2]], rows[b], sems[b])
            return 0

        lax.fori_loop(0, PH // 2, _pair_body, 0)
    plsc.subcore_barrier()

    o0 = c * N_PAD + r0
    pltpu.sync_copy(acc_sh.at[pl.ds(r0, ROWS_PT)], part_hbm.at[pl.ds(o0, ROWS_PT)])
    if with_deg:
        pltpu.sync_copy(deg_sh.at[pl.ds(r0, ROWS_PT)], degp_hbm.at[pl.ds(o0, ROWS_PT)])


@functools.cache
def _sc_kernels():
    mesh = plsc.VectorSubcoreMesh(core_axis_name="c", subcore_axis_name="s")

    @functools.partial(
        pl.kernel,
        out_type=[jax.ShapeDtypeStruct((2 * N_PAD, D), jnp.float32),
                  jax.ShapeDtypeStruct((2 * N_PAD,), jnp.float32)],
        mesh=mesh,
        scratch_types=[
            pltpu.VMEM_SHARED((N_PAD, D), jnp.float32),
            pltpu.VMEM_SHARED((N_PAD,), jnp.float32),
            pltpu.VMEM((PH, CH), jnp.int32),
            pltpu.VMEM((PH, CH), jnp.int32),
            pltpu.VMEM((CH, D), jnp.float32),
            pltpu.VMEM((CH, D), jnp.float32),
            pltpu.VMEM((CH,), jnp.float32),
            pltpu.SemaphoreType.DMA,
            pltpu.SemaphoreType.DMA,
        ],
    )
    def sc_agg_deg(y_hbm, src_hbm, dst_hbm, ones_hbm,
                   part_hbm, degp_hbm,
                   acc_sh, deg_sh, src_all, dst_all, rows0, rows1, ones_v,
                   sem0, sem1):
        _sc_body(y_hbm, src_hbm, dst_hbm, ones_hbm,
                 part_hbm, degp_hbm,
                 acc_sh, deg_sh, src_all, dst_all, rows0, rows1, ones_v,
                 sem0, sem1, True)

    @functools.partial(
        pl.kernel,
        out_type=[jax.ShapeDtypeStruct((2 * N_PAD, D), jnp.float32)],
        mesh=mesh,
        scratch_types=[
            pltpu.VMEM_SHARED((N_PAD, D), jnp.float32),
            pltpu.VMEM((PH, CH), jnp.int32),
            pltpu.VMEM((PH, CH), jnp.int32),
            pltpu.VMEM((CH, D), jnp.float32),
            pltpu.VMEM((CH, D), jnp.float32),
            pltpu.SemaphoreType.DMA,
            pltpu.SemaphoreType.DMA,
        ],
    )
    def sc_agg(y_hbm, src_hbm, dst_hbm,
               part_hbm,
               acc_sh, src_all, dst_all, rows0, rows1, sem0, sem1):
        _sc_body(y_hbm, src_hbm, dst_hbm, None, part_hbm, None,
                 acc_sh, None, src_all, dst_all, rows0, rows1, None,
                 sem0, sem1, False)

    return sc_agg_deg, sc_agg


BN = 1000
BNP = 1024


def _comb_body(p_ref, dg0_ref, dg1_ref, x_ref, b_ref, wm1_ref, wr1_ref,
               wr2_ref, h_ref, r2_ref, invb_ref):
    invd_row = 1.0 / jnp.maximum(dg0_ref[...] + dg1_ref[...], 1.0)
    invd = jnp.swapaxes(invd_row, 0, 1)
    agg = (p_ref[0] + p_ref[1]) * invd
    h = jnp.maximum(
        jnp.dot(agg, wm1_ref[...], preferred_element_type=jnp.float32)
        + jnp.dot(x_ref[...], wr1_ref[...], preferred_element_type=jnp.float32)
        + b_ref[...], 0.0)
    h_ref[...] = h
    r2_ref[...] = jnp.dot(h, wr2_ref[...], preferred_element_type=jnp.float32)
    invb_ref[...] = jnp.broadcast_to(invd, (BNP, D))


_comb = pl.pallas_call(
    _comb_body,
    grid=(N_PAD // BNP,),
    in_specs=[pl.BlockSpec((2, BNP, D), lambda i: (0, i, 0)),
              pl.BlockSpec((1, BNP), lambda i: (0, i)),
              pl.BlockSpec((1, BNP), lambda i: (0, i)),
              pl.BlockSpec((BNP, D), lambda i: (i, 0)),
              pl.BlockSpec((1, D), lambda i: (0, 0)),
              pl.BlockSpec((D, D), lambda i: (0, 0)),
              pl.BlockSpec((D, D), lambda i: (0, 0)),
              pl.BlockSpec((D, D), lambda i: (0, 0))],
    out_specs=[pl.BlockSpec((BNP, D), lambda i: (i, 0)),
               pl.BlockSpec((BNP, D), lambda i: (i, 0)),
               pl.BlockSpec((BNP, D), lambda i: (i, 0))],
    out_shape=[jax.ShapeDtypeStruct((N_PAD, D), jnp.float32),
               jax.ShapeDtypeStruct((N_PAD, D), jnp.float32),
               jax.ShapeDtypeStruct((N_PAD, D), jnp.float32)],
)


def _fin_body(p_ref, invb_ref, r_ref, b_ref, wm2_ref, o_ref):
    agg = (p_ref[0] + p_ref[1]) * invb_ref[...]
    o_ref[...] = jnp.maximum(
        jnp.dot(agg, wm2_ref[...], preferred_element_type=jnp.float32)
        + r_ref[...] + b_ref[...], 0.0)


_fin = pl.pallas_call(
    _fin_body,
    grid=(N // BN,),
    in_specs=[pl.BlockSpec((2, BN, D), lambda i: (0, i, 0)),
              pl.BlockSpec((BN, D), lambda i: (i, 0)),
              pl.BlockSpec((BN, D), lambda i: (i, 0)),
              pl.BlockSpec((1, D), lambda i: (0, 0)),
              pl.BlockSpec((D, D), lambda i: (0, 0))],
    out_specs=pl.BlockSpec((BN, D), lambda i: (i, 0)),
    out_shape=jax.ShapeDtypeStruct((N, D), jnp.float32),
)


def kernel(x, edge_index, W_msg1, W_root1, b1, W_msg2, W_root2, b2):
    pad_rows = (E_PAD - E) // CH
    src2 = jax.lax.slice(edge_index, (0, 0), (1, E)).reshape(E // CH, CH)
    dst2 = jax.lax.slice(edge_index, (1, 0), (2, E)).reshape(E // CH, CH)
    pad_iota = jax.lax.broadcasted_iota(jnp.int32, (pad_rows, CH), 1)
    src = jnp.concatenate([src2, pad_iota], axis=0)
    dst = jnp.concatenate([dst2, N + pad_iota], axis=0)
    ones = jnp.ones((CH,), jnp.float32)
    x_pad = jnp.pad(x, ((0, N_PAD - N), (0, 0)))

    sc_agg_deg, sc_agg = _sc_kernels()
    part1, degp1 = sc_agg_deg(x, src, dst, ones)
    p1 = part1.reshape(2, N_PAD, D)
    dg0 = degp1[:N_PAD].reshape(1, N_PAD)
    dg1 = degp1[N_PAD:].reshape(1, N_PAD)
    h, r2, invb = _comb(p1, dg0, dg1, x_pad, b1.reshape(1, D),
                        W_msg1, W_root1, W_root2)
    (part2,) = sc_agg(h, src, dst)
    out = _fin(part2.reshape(2, N_PAD, D), invb, r2, b2.reshape(1, D), W_msg2)
    return out
